# baseline (speedup 1.0000x reference)
import functools

import jax
import jax.numpy as jnp
from jax import lax
from jax.experimental import pallas as pl
from jax.experimental.pallas import tpu as pltpu
from jax.experimental.pallas import tpu_sc as plsc

S = 200
D = 64
B = 1024
NC = 2
NS = 16
NW = NC * NS
SEQ_PER_W = B // NW
SPLIT0 = 128
SPLIT1 = S - SPLIT0


def _body(x_hbm, tok_hbm, pos_hbm, out_hbm, idx_v, rows_v, pos_v, sem):
    wid = lax.axis_index("s") * NC + lax.axis_index("c")
    pltpu.sync_copy(pos_hbm, pos_v)

    def seq_body(i, carry):
        seq = wid * SEQ_PER_W + i
        pltpu.sync_copy(x_hbm.at[seq], idx_v)
        cp0 = pltpu.async_copy(
            tok_hbm.at[idx_v.at[pl.ds(0, SPLIT0)]],
            rows_v.at[pl.ds(0, SPLIT0)], sem)
        cp1 = pltpu.async_copy(
            tok_hbm.at[idx_v.at[pl.ds(SPLIT0, SPLIT1)]],
            rows_v.at[pl.ds(SPLIT0, SPLIT1)], sem)
        cp0.wait()
        cp1.wait()

        def row_body(r, c2):
            for c in range(D // 16):
                sl = pl.ds(c * 16, 16)
                rows_v[r, sl] = rows_v[r, sl] + pos_v[r, sl]
            return c2

        lax.fori_loop(0, S, row_body, 0)
        pltpu.sync_copy(rows_v, out_hbm.at[seq])
        return carry

    lax.fori_loop(0, SEQ_PER_W, seq_body, 0)


@functools.partial(
    pl.kernel,
    mesh=plsc.VectorSubcoreMesh(core_axis_name="c", subcore_axis_name="s"),
    compiler_params=pltpu.CompilerParams(use_tc_tiling_on_sc=False),
    out_type=jax.ShapeDtypeStruct((B, S, D), jnp.float32),
    scratch_types=[
        pltpu.VMEM((S,), jnp.int32),
        pltpu.VMEM((S, D), jnp.float32),
        pltpu.VMEM((S, D), jnp.float32),
        pltpu.SemaphoreType.DMA,
    ],
)
def _embed(x_hbm, tok_hbm, pos_hbm, out_hbm, idx_v, rows_v, pos_v, sem):
    _body(x_hbm, tok_hbm, pos_hbm, out_hbm, idx_v, rows_v, pos_v, sem)


@jax.jit
def kernel(x, token_table, pos_table):
    return _embed(x, token_table, pos_table)


# speedup vs baseline: 1.1029x; 1.1029x over previous
import functools

import jax
import jax.numpy as jnp
from jax import lax
from jax.experimental import pallas as pl
from jax.experimental.pallas import tpu as pltpu
from jax.experimental.pallas import tpu_sc as plsc

S = 200
D = 64
B = 1024
NC = 2
NS = 16
NW = NC * NS
SEQ_PER_W = B // NW
SPLIT0 = 128
SPLIT1 = S - SPLIT0


def _body(x_hbm, tab_hbm, pos_hbm, out_hbm, idx_v, stag_v, rows_v, pos_v, sem):
    wid = lax.axis_index("s") * NC + lax.axis_index("c")
    pltpu.sync_copy(pos_hbm, pos_v)

    def seq_body(i, carry):
        seq = wid * SEQ_PER_W + i
        pltpu.sync_copy(x_hbm.at[pl.ds(seq * S, S)], idx_v)
        cp0 = pltpu.async_copy(
            tab_hbm.at[idx_v.at[pl.ds(0, SPLIT0)]],
            stag_v.at[pl.ds(0, SPLIT0)], sem)
        cp1 = pltpu.async_copy(
            tab_hbm.at[idx_v.at[pl.ds(SPLIT0, SPLIT1)]],
            stag_v.at[pl.ds(SPLIT0, SPLIT1)], sem)
        cp0.wait()
        cp1.wait()

        def row_body(r, c2):
            for c in range(D // 16):
                sl = pl.ds(c * 16, 16)
                rows_v[r, sl] = stag_v[r, sl] + pos_v[r, sl]
            return c2

        lax.fori_loop(0, S, row_body, 0)
        pltpu.sync_copy(rows_v, out_hbm.at[seq])
        return carry

    lax.fori_loop(0, SEQ_PER_W, seq_body, 0)


@functools.partial(
    pl.kernel,
    mesh=plsc.VectorSubcoreMesh(core_axis_name="c", subcore_axis_name="s"),
    compiler_params=pltpu.CompilerParams(use_tc_tiling_on_sc=True),
    out_type=jax.ShapeDtypeStruct((B, S, D), jnp.float32),
    scratch_types=[
        pltpu.VMEM((S,), jnp.int32),
        pltpu.VMEM((S, 2 * D), jnp.float32),
        pltpu.VMEM((S, D), jnp.float32),
        pltpu.VMEM((S, D), jnp.float32),
        pltpu.SemaphoreType.DMA,
    ],
)
def _embed(x_hbm, tab_hbm, pos_hbm, out_hbm, idx_v, stag_v, rows_v, pos_v, sem):
    _body(x_hbm, tab_hbm, pos_hbm, out_hbm, idx_v, stag_v, rows_v, pos_v, sem)


@jax.jit
def kernel(x, token_table, pos_table):
    tabp = jnp.pad(token_table, ((0, 0), (0, D)))
    return _embed(x.reshape(B * S), tabp, pos_table)
